# baseline (speedup 1.0000x reference)
import jax
import jax.numpy as jnp
from jax import lax
from jax.experimental import pallas as pl

_TOPK = 13
_EPS_IN = 1e-09
_IOU_EPS = 1e-7


def _assign_body(ps_ref, pbt_ref, apt_ref, gl_ref, gb_ref, mg_ref,
                 tb_ref, ts_ref, fg_ref):
    ps = ps_ref[0]
    pbt = pbt_ref[0]
    apt = apt_ref[...]
    gl = gl_ref[0]
    gb = gb_ref[0]
    mg = mg_ref[0]
    n = gb.shape[0]
    A, nc = ps.shape

    ax = apt[0:1, :]
    ay = apt[1:2, :]
    px1 = pbt[0:1, :]
    py1 = pbt[1:2, :]
    px2 = pbt[2:3, :]
    py2 = pbt[3:4, :]
    gx1 = gb[:, 0:1]
    gy1 = gb[:, 1:2]
    gx2 = gb[:, 2:3]
    gy2 = gb[:, 3:4]

    d_min = jnp.minimum(jnp.minimum(ax - gx1, ay - gy1),
                        jnp.minimum(gx2 - ax, gy2 - ay))
    mask0 = jnp.where(d_min > _EPS_IN, 1.0, 0.0) * mg

    iw = jnp.maximum(jnp.minimum(gx2, px2) - jnp.maximum(gx1, px1), 0.0)
    ih = jnp.maximum(jnp.minimum(gy2, py2) - jnp.maximum(gy1, py1), 0.0)
    inter = iw * ih
    area_g = (gx2 - gx1) * (gy2 - gy1)
    area_p = (px2 - px1) * (py2 - py1)
    union = area_g + area_p - inter + _IOU_EPS
    ov = jnp.maximum(inter / union, 0.0)

    lab_onehot = (gl == lax.broadcasted_iota(jnp.int32, (1, nc), 1)
                  ).astype(jnp.float32)
    bs = lax.dot_general(lab_onehot, ps, (((1,), (1,)), ((), ())),
                         preferred_element_type=jnp.float32,
                         precision=lax.Precision.HIGHEST)

    o3 = ov * ov * ov
    align = bs * (o3 * o3) * mask0

    iota_a = lax.broadcasted_iota(jnp.int32, (n, A), 1)
    v = align
    for _ in range(_TOPK):
        m = jnp.max(v, axis=1, keepdims=True)
        idx = jnp.min(jnp.where(v == m, iota_a, A), axis=1, keepdims=True)
        v = jnp.where(iota_a == idx, -1.0, v)

    mask_pos = jnp.where(v < 0.0, mask0, 0.0)
    fg = jnp.sum(mask_pos, axis=0, keepdims=True)
    mx = jnp.max(align, axis=0, keepdims=True)
    is_max = (align == mx).astype(jnp.float32)
    mask_pos = jnp.where(fg > 1.0, mask_pos * is_max, mask_pos)
    fg = jnp.sum(mask_pos, axis=0, keepdims=True)

    colmax = jnp.max(mask_pos, axis=0, keepdims=True)
    iota_n = lax.broadcasted_iota(jnp.int32, (n, A), 0)
    tgi = jnp.min(jnp.where(mask_pos == colmax, iota_n, n),
                  axis=0, keepdims=True)
    sel_n = (iota_n == tgi).astype(jnp.float32)

    am = align * mask_pos
    pam = jnp.max(am, axis=0, keepdims=True)
    pam_col = jnp.transpose(pam, (1, 0))

    sel_b = sel_n.astype(jnp.bfloat16)
    gb_hi = gb.astype(jnp.bfloat16)
    gb_r1 = gb - gb_hi.astype(jnp.float32)
    gb_mid = gb_r1.astype(jnp.bfloat16)
    gb_lo = (gb_r1 - gb_mid.astype(jnp.float32)).astype(jnp.bfloat16)
    dims = (((0,), (0,)), ((), ()))
    tb = (lax.dot_general(sel_b, gb_hi, dims,
                          preferred_element_type=jnp.float32)
          + lax.dot_general(sel_b, gb_mid, dims,
                            preferred_element_type=jnp.float32)
          + lax.dot_general(sel_b, gb_lo, dims,
                            preferred_element_type=jnp.float32))

    ts0 = lax.dot_general(sel_b, lab_onehot.astype(jnp.bfloat16), dims,
                          preferred_element_type=jnp.float32)
    ts = ts0 * pam_col

    tb_ref[0] = tb
    ts_ref[0] = ts
    fg_ref[0] = fg


def kernel(pd_scores, pd_bboxes, anc_points, gt_labels, gt_bboxes, mask_gt):
    b, A, nc = pd_scores.shape
    n = gt_bboxes.shape[1]
    pbt = jnp.transpose(pd_bboxes, (0, 2, 1))
    apt = jnp.transpose(anc_points, (1, 0))
    gl = gt_labels.astype(jnp.int32)
    mg = mask_gt.astype(jnp.float32)

    tb, ts, fg = pl.pallas_call(
        _assign_body,
        grid=(b,),
        in_specs=[
            pl.BlockSpec((1, A, nc), lambda i: (i, 0, 0)),
            pl.BlockSpec((1, 4, A), lambda i: (i, 0, 0)),
            pl.BlockSpec((2, A), lambda i: (0, 0)),
            pl.BlockSpec((1, n, 1), lambda i: (i, 0, 0)),
            pl.BlockSpec((1, n, 4), lambda i: (i, 0, 0)),
            pl.BlockSpec((1, n, 1), lambda i: (i, 0, 0)),
        ],
        out_specs=[
            pl.BlockSpec((1, A, 4), lambda i: (i, 0, 0)),
            pl.BlockSpec((1, A, nc), lambda i: (i, 0, 0)),
            pl.BlockSpec((1, 1, A), lambda i: (i, 0, 0)),
        ],
        out_shape=[
            jax.ShapeDtypeStruct((b, A, 4), jnp.float32),
            jax.ShapeDtypeStruct((b, A, nc), jnp.float32),
            jax.ShapeDtypeStruct((b, 1, A), jnp.float32),
        ],
    )(pd_scores, pbt, apt, gl, gt_bboxes, mg)
    return (tb, ts, fg[:, 0, :].astype(bool))


# speedup vs baseline: 1.0003x; 1.0003x over previous
import jax
import jax.numpy as jnp
from jax import lax
from jax.experimental import pallas as pl
from jax.experimental.pallas import tpu as pltpu

_TOPK = 13
_EPS_IN = 1e-09
_IOU_EPS = 1e-7


def _assign_body(ps_ref, pbt_ref, apt_ref, gl_ref, gb_ref, mg_ref,
                 tb_ref, ts_ref, fg_ref):
    ps = ps_ref[0]
    pbt = pbt_ref[0]
    apt = apt_ref[...]
    gl = gl_ref[0]
    gb = gb_ref[0]
    mg = mg_ref[0]
    n = gb.shape[0]
    A, nc = ps.shape

    ax = apt[0:1, :]
    ay = apt[1:2, :]
    px1 = pbt[0:1, :]
    py1 = pbt[1:2, :]
    px2 = pbt[2:3, :]
    py2 = pbt[3:4, :]
    gx1 = gb[:, 0:1]
    gy1 = gb[:, 1:2]
    gx2 = gb[:, 2:3]
    gy2 = gb[:, 3:4]

    d_min = jnp.minimum(jnp.minimum(ax - gx1, ay - gy1),
                        jnp.minimum(gx2 - ax, gy2 - ay))
    mask0 = jnp.where(d_min > _EPS_IN, 1.0, 0.0) * mg

    iw = jnp.maximum(jnp.minimum(gx2, px2) - jnp.maximum(gx1, px1), 0.0)
    ih = jnp.maximum(jnp.minimum(gy2, py2) - jnp.maximum(gy1, py1), 0.0)
    inter = iw * ih
    area_g = (gx2 - gx1) * (gy2 - gy1)
    area_p = (px2 - px1) * (py2 - py1)
    union = area_g + area_p - inter + _IOU_EPS
    ov = jnp.maximum(inter / union, 0.0)

    lab_onehot = (gl == lax.broadcasted_iota(jnp.int32, (1, nc), 1)
                  ).astype(jnp.float32)
    bs = lax.dot_general(lab_onehot, ps, (((1,), (1,)), ((), ())),
                         preferred_element_type=jnp.float32,
                         precision=lax.Precision.HIGHEST)

    o3 = ov * ov * ov
    align = bs * (o3 * o3) * mask0

    iota_a = lax.broadcasted_iota(jnp.int32, (n, A), 1)
    v = align
    for _ in range(_TOPK):
        m = jnp.max(v, axis=1, keepdims=True)
        idx = jnp.min(jnp.where(v == m, iota_a, A), axis=1, keepdims=True)
        v = jnp.where(iota_a == idx, -1.0, v)

    mask_pos = jnp.where(v < 0.0, mask0, 0.0)
    fg = jnp.sum(mask_pos, axis=0, keepdims=True)
    mx = jnp.max(align, axis=0, keepdims=True)
    is_max = (align == mx).astype(jnp.float32)
    mask_pos = jnp.where(fg > 1.0, mask_pos * is_max, mask_pos)
    fg = jnp.sum(mask_pos, axis=0, keepdims=True)

    colmax = jnp.max(mask_pos, axis=0, keepdims=True)
    iota_n = lax.broadcasted_iota(jnp.int32, (n, A), 0)
    tgi = jnp.min(jnp.where(mask_pos == colmax, iota_n, n),
                  axis=0, keepdims=True)
    sel_n = (iota_n == tgi).astype(jnp.float32)

    am = align * mask_pos
    pam = jnp.max(am, axis=0, keepdims=True)
    pam_col = jnp.transpose(pam, (1, 0))

    sel_b = sel_n.astype(jnp.bfloat16)
    gb_hi = gb.astype(jnp.bfloat16)
    gb_r1 = gb - gb_hi.astype(jnp.float32)
    gb_mid = gb_r1.astype(jnp.bfloat16)
    gb_lo = (gb_r1 - gb_mid.astype(jnp.float32)).astype(jnp.bfloat16)
    dims = (((0,), (0,)), ((), ()))
    tb = (lax.dot_general(sel_b, gb_hi, dims,
                          preferred_element_type=jnp.float32)
          + lax.dot_general(sel_b, gb_mid, dims,
                            preferred_element_type=jnp.float32)
          + lax.dot_general(sel_b, gb_lo, dims,
                            preferred_element_type=jnp.float32))

    ts0 = lax.dot_general(sel_b, lab_onehot.astype(jnp.bfloat16), dims,
                          preferred_element_type=jnp.float32)
    ts = ts0 * pam_col

    tb_ref[0] = tb
    ts_ref[0] = ts
    fg_ref[0] = fg


def kernel(pd_scores, pd_bboxes, anc_points, gt_labels, gt_bboxes, mask_gt):
    b, A, nc = pd_scores.shape
    n = gt_bboxes.shape[1]
    pbt = jnp.transpose(pd_bboxes, (0, 2, 1))
    apt = jnp.transpose(anc_points, (1, 0))
    gl = gt_labels.astype(jnp.int32)
    mg = mask_gt.astype(jnp.float32)

    tb, ts, fg = pl.pallas_call(
        _assign_body,
        grid=(b,),
        in_specs=[
            pl.BlockSpec((1, A, nc), lambda i: (i, 0, 0)),
            pl.BlockSpec((1, 4, A), lambda i: (i, 0, 0)),
            pl.BlockSpec((2, A), lambda i: (0, 0)),
            pl.BlockSpec((1, n, 1), lambda i: (i, 0, 0)),
            pl.BlockSpec((1, n, 4), lambda i: (i, 0, 0)),
            pl.BlockSpec((1, n, 1), lambda i: (i, 0, 0)),
        ],
        out_specs=[
            pl.BlockSpec((1, A, 4), lambda i: (i, 0, 0)),
            pl.BlockSpec((1, A, nc), lambda i: (i, 0, 0)),
            pl.BlockSpec((1, 1, A), lambda i: (i, 0, 0)),
        ],
        out_shape=[
            jax.ShapeDtypeStruct((b, A, 4), jnp.float32),
            jax.ShapeDtypeStruct((b, A, nc), jnp.float32),
            jax.ShapeDtypeStruct((b, 1, A), jnp.float32),
        ],
        compiler_params=pltpu.CompilerParams(
            vmem_limit_bytes=100 * 1024 * 1024,
        ),
    )(pd_scores, pbt, apt, gl, gt_bboxes, mg)
    return (tb, ts, fg[:, 0, :].astype(bool))
